# Initial kernel scaffold; baseline (speedup 1.0000x reference)
import jax
import jax.numpy as jnp
from jax import lax
from jax.experimental import pallas as pl
from jax.experimental.pallas import tpu as pltpu
from jax.experimental.pallas import tpu_sc as plsc

N = 10000
NPAD = 10240
D = 128
DO = 64
E = 320000
NC, NS = 2, 16
NW = NC * NS
CH = 128
RPT = NPAD // NS


def _ceil_to(n, m):
    return ((n + m - 1) // m) * m


EP1 = _ceil_to(E, NW * CH)
EP2 = _ceil_to(E + N, NW * CH)
CH1 = EP1 // (NW * CH)
CH2 = EP2 // (NW * CH)
CH2A = EP2 // (NS * CH)

_MESH = plsc.VectorSubcoreMesh(
    core_axis_name="c", subcore_axis_name="s", num_cores=NC, num_subcores=NS
)


def _deg_body(dst_hbm, zeros_hbm, out_hbm, didx, ones_v, acc):
    c = lax.axis_index("c")
    s = lax.axis_index("s")
    w = c * NS + s
    r0 = s * RPT
    pltpu.sync_copy(zeros_hbm.at[pl.ds(r0, RPT)], acc.at[pl.ds(r0, RPT)])
    for g in range(8):
        ones_v[pl.ds(g * 16, 16)] = jnp.full((16,), 1.0, jnp.float32)
    plsc.subcore_barrier()

    def body(i, carry):
        base = (w * CH1 + i) * CH
        pltpu.sync_copy(dst_hbm.at[pl.ds(base, CH)], didx)
        pltpu.sync_copy(ones_v, acc.at[didx], add=True)
        return carry

    lax.fori_loop(0, CH1, body, 0)
    plsc.subcore_barrier()
    pltpu.sync_copy(acc.at[pl.ds(r0, RPT)], out_hbm.at[c, pl.ds(r0, RPT)])


_deg_call = pl.kernel(
    _deg_body,
    out_type=jax.ShapeDtypeStruct((NC, NPAD), jnp.float32),
    mesh=_MESH,
    scratch_types=[
        pltpu.VMEM((CH,), jnp.int32),
        pltpu.VMEM((CH,), jnp.float32),
        pltpu.VMEM_SHARED((NPAD,), jnp.float32),
    ],
)


def _gcn_body(src_hbm, dst_hbm, table_hbm, zeros_hbm, out_hbm, sidx, didx,
              rows, acc, sem):
    c = lax.axis_index("c")
    s = lax.axis_index("s")
    w = c * NS + s
    r0 = s * RPT
    pltpu.sync_copy(zeros_hbm.at[pl.ds(r0, RPT)], acc.at[pl.ds(r0, RPT)])
    plsc.subcore_barrier()

    def body(i, carry):
        base = (w * CH1 + i) * CH
        pltpu.sync_copy(src_hbm.at[pl.ds(base, CH)], sidx)
        pltpu.sync_copy(dst_hbm.at[pl.ds(base, CH)], didx)
        pltpu.async_copy(table_hbm.at[sidx], rows, sem).wait()
        pltpu.sync_copy(rows, acc.at[didx], add=True)
        return carry

    lax.fori_loop(0, CH1, body, 0)
    plsc.subcore_barrier()
    pltpu.sync_copy(acc.at[pl.ds(r0, RPT)], out_hbm.at[c, pl.ds(r0, RPT)])


def _make_gcn(dd):
    return pl.kernel(
        _gcn_body,
        out_type=jax.ShapeDtypeStruct((NC, NPAD, dd), jnp.float32),
        mesh=_MESH,
        scratch_types=[
            pltpu.VMEM((CH,), jnp.int32),
            pltpu.VMEM((CH,), jnp.int32),
            pltpu.VMEM((CH, dd), jnp.float32),
            pltpu.VMEM_SHARED((NPAD, dd), jnp.float32),
            pltpu.SemaphoreType.DMA,
        ],
    )


_gcn128 = _make_gcn(D)
_gcn64 = _make_gcn(DO)


def _gata_body(src_hbm, dst_hbm, al_hbm, gmax_hbm, zeros4_hbm, denom_hbm,
               ex_hbm, sidx, didx, tab, gmaxv, exv, dacc):
    c = lax.axis_index("c")
    s = lax.axis_index("s")

    @pl.when(c == 0)
    def _():
        r0 = s * RPT
        pltpu.sync_copy(zeros4_hbm.at[pl.ds(r0, RPT)], dacc.at[pl.ds(r0, RPT)])
        pltpu.sync_copy(al_hbm, tab)
        pltpu.sync_copy(gmax_hbm, gmaxv)
        plsc.subcore_barrier()
        lanes = jnp.arange(16, dtype=jnp.int32)

        def body(i, carry):
            base = (s * CH2A + i) * CH
            pltpu.sync_copy(src_hbm.at[pl.ds(base, CH)], sidx)
            pltpu.sync_copy(dst_hbm.at[pl.ds(base, CH)], didx)
            for g in range(8):
                sv = sidx[pl.ds(g * 16, 16)]
                dv = didx[pl.ds(g * 16, 16)]
                lid = lanes + g * 16
                for h in range(4):
                    hn = jnp.full((16,), h, jnp.int32)
                    a1 = plsc.load_gather(tab, [sv, hn])
                    a2 = plsc.load_gather(
                        tab, [dv, jnp.full((16,), 4 + h, jnp.int32)])
                    sc = a1 + a2
                    sc = jnp.maximum(sc, sc * 0.2) - gmaxv[h]
                    plsc.store_scatter(exv, [lid, hn], jnp.exp(sc))
            pltpu.sync_copy(exv, ex_hbm.at[pl.ds(base, CH)])
            pltpu.sync_copy(exv, dacc.at[didx], add=True)
            return carry

        lax.fori_loop(0, CH2A, body, 0)
        plsc.subcore_barrier()
        pltpu.sync_copy(dacc.at[pl.ds(r0, RPT)], denom_hbm.at[pl.ds(r0, RPT)])


_gata_call = pl.kernel(
    _gata_body,
    out_type=(
        jax.ShapeDtypeStruct((NPAD, 4), jnp.float32),
        jax.ShapeDtypeStruct((EP2, 4), jnp.float32),
    ),
    mesh=_MESH,
    scratch_types=[
        pltpu.VMEM((CH,), jnp.int32),
        pltpu.VMEM((CH,), jnp.int32),
        pltpu.VMEM((NPAD, 8), jnp.float32),
        pltpu.VMEM((16,), jnp.float32),
        pltpu.VMEM((CH, 4), jnp.float32),
        pltpu.VMEM_SHARED((NPAD, 4), jnp.float32),
    ],
)


def _gatb_body(src_hbm, dst_hbm, ex_hbm, denom_hbm, g_hbm, zeros64_hbm,
               out_hbm, sidx, didx, dtab, exv, wbuf, rows, orows, acc, sem):
    c = lax.axis_index("c")
    s = lax.axis_index("s")
    w = c * NS + s
    r0 = s * RPT
    pltpu.sync_copy(zeros64_hbm.at[pl.ds(r0, RPT)], acc.at[pl.ds(r0, RPT)])
    pltpu.sync_copy(denom_hbm, dtab)
    plsc.subcore_barrier()
    lanes = jnp.arange(16, dtype=jnp.int32)

    def body(i, carry):
        base = (w * CH2 + i) * CH
        pltpu.sync_copy(src_hbm.at[pl.ds(base, CH)], sidx)
        pltpu.sync_copy(dst_hbm.at[pl.ds(base, CH)], didx)
        pltpu.sync_copy(ex_hbm.at[pl.ds(base, CH)], exv)
        gat = pltpu.async_copy(g_hbm.at[sidx], rows, sem)
        for g in range(8):
            dv = didx[pl.ds(g * 16, 16)]
            lid = lanes + g * 16
            for h in range(4):
                hn = jnp.full((16,), h, jnp.int32)
                dn = plsc.load_gather(dtab, [dv, hn])
                exg = plsc.load_gather(exv, [lid, hn])
                wv = exg / (dn + 1e-16) * 0.25
                plsc.store_scatter(wbuf, [lid, hn], wv)
        gat.wait()

        def ebody(k, carry2):
            for j in range(4):
                o = rows[k, pl.ds(j * 16, 16)] * wbuf[k, 0]
                for h in range(1, 4):
                    o = o + rows[k, pl.ds(h * 64 + j * 16, 16)] * wbuf[k, h]
                orows[k, pl.ds(j * 16, 16)] = o
            return carry2

        lax.fori_loop(0, CH, ebody, 0)
        pltpu.sync_copy(orows, acc.at[didx], add=True)
        return carry

    lax.fori_loop(0, CH2, body, 0)
    plsc.subcore_barrier()
    pltpu.sync_copy(acc.at[pl.ds(r0, RPT)], out_hbm.at[c, pl.ds(r0, RPT)])


_gatb_call = pl.kernel(
    _gatb_body,
    out_type=jax.ShapeDtypeStruct((NC, NPAD, DO), jnp.float32),
    mesh=_MESH,
    scratch_types=[
        pltpu.VMEM((CH,), jnp.int32),
        pltpu.VMEM((CH,), jnp.int32),
        pltpu.VMEM((NPAD, 4), jnp.float32),
        pltpu.VMEM((CH, 4), jnp.float32),
        pltpu.VMEM((CH, 4), jnp.float32),
        pltpu.VMEM((CH, 4 * DO), jnp.float32),
        pltpu.VMEM((CH, DO), jnp.float32),
        pltpu.VMEM_SHARED((NPAD, DO), jnp.float32),
        pltpu.SemaphoreType.DMA,
    ],
)


def _edge_body(src_hbm, dst_hbm, g1_hbm, g2_hbm, p2_hbm, pb2_hbm, out_hbm,
               sidx, didx, r1, r2, p2v, pbv, tv, predv, sem1, sem2):
    c = lax.axis_index("c")
    s = lax.axis_index("s")
    w = c * NS + s
    pltpu.sync_copy(p2_hbm, p2v)
    pltpu.sync_copy(pb2_hbm, pbv)

    def body(i, carry):
        base = (w * CH1 + i) * CH
        pltpu.sync_copy(src_hbm.at[pl.ds(base, CH)], sidx)
        pltpu.sync_copy(dst_hbm.at[pl.ds(base, CH)], didx)
        d1 = pltpu.async_copy(g1_hbm.at[sidx], r1, sem1)
        d2 = pltpu.async_copy(g2_hbm.at[didx], r2, sem2)
        d1.wait()
        d2.wait()
        p2regs = [p2v[pl.ds(j * 16, 16)] for j in range(8)]
        pb2s = pbv[0]

        def ebody(k, carry2):
            z = jnp.maximum(r1[k, pl.ds(0, 16)] + r2[k, pl.ds(0, 16)], 0.0)
            acc = z * p2regs[0]
            for j in range(1, 8):
                z = jnp.maximum(
                    r1[k, pl.ds(j * 16, 16)] + r2[k, pl.ds(j * 16, 16)], 0.0)
                acc = acc + z * p2regs[j]
            tv[k] = jnp.sum(acc)
            return carry2

        lax.fori_loop(0, CH, ebody, 0)
        for g in range(8):
            t = tv[pl.ds(g * 16, 16)] + pb2s
            predv[pl.ds(g * 16, 16)] = 1.0 / (1.0 + jnp.exp(-t))
        pltpu.sync_copy(predv, out_hbm.at[pl.ds(base, CH)])
        return carry

    lax.fori_loop(0, CH1, body, 0)


_edge_call = pl.kernel(
    _edge_body,
    out_type=jax.ShapeDtypeStruct((EP1,), jnp.float32),
    mesh=_MESH,
    scratch_types=[
        pltpu.VMEM((CH,), jnp.int32),
        pltpu.VMEM((CH,), jnp.int32),
        pltpu.VMEM((CH, D), jnp.float32),
        pltpu.VMEM((CH, D), jnp.float32),
        pltpu.VMEM((D,), jnp.float32),
        pltpu.VMEM((16,), jnp.float32),
        pltpu.VMEM((CH,), jnp.float32),
        pltpu.VMEM((CH,), jnp.float32),
        pltpu.SemaphoreType.DMA,
        pltpu.SemaphoreType.DMA,
    ],
)


BLK = 256
GRID = NPAD // BLK


def _dinv_block(deg_ref, i):
    db = deg_ref[:, pl.ds(i * BLK, BLK)]
    d = db[0, :] + db[1, :] + 1.0
    return lax.rsqrt(jnp.maximum(d, 1.0))


def _tc0_body(x_ref, deg_ref, w_ref, o_ref):
    dinv = _dinv_block(deg_ref, pl.program_id(0))
    p = jnp.dot(x_ref[...], w_ref[...], preferred_element_type=jnp.float32)
    o_ref[...] = p * dinv[:, None]


_tc0_call = pl.pallas_call(
    _tc0_body,
    grid=(GRID,),
    in_specs=[
        pl.BlockSpec((BLK, D), lambda i: (i, 0)),
        pl.BlockSpec((NC, NPAD), lambda i: (0, 0)),
        pl.BlockSpec((D, D), lambda i: (0, 0)),
    ],
    out_specs=pl.BlockSpec((BLK, D), lambda i: (i, 0)),
    out_shape=jax.ShapeDtypeStruct((NPAD, D), jnp.float32),
)


def _tcmid_body(s_ref, p_ref, deg_ref, b_ref, w_ref, o_ref):
    dinv = _dinv_block(deg_ref, pl.program_id(0))
    h = s_ref[0] + s_ref[1] + p_ref[...]
    h = jnp.maximum(h * dinv[:, None] + b_ref[...], 0.0)
    o_ref[...] = jnp.dot(h, w_ref[...],
                         preferred_element_type=jnp.float32) * dinv[:, None]


def _make_tcmid(di, do):
    return pl.pallas_call(
        _tcmid_body,
        grid=(GRID,),
        in_specs=[
            pl.BlockSpec((NC, BLK, di), lambda i: (0, i, 0)),
            pl.BlockSpec((BLK, di), lambda i: (i, 0)),
            pl.BlockSpec((NC, NPAD), lambda i: (0, 0)),
            pl.BlockSpec((1, di), lambda i: (0, 0)),
            pl.BlockSpec((di, do), lambda i: (0, 0)),
        ],
        out_specs=pl.BlockSpec((BLK, do), lambda i: (i, 0)),
        out_shape=jax.ShapeDtypeStruct((NPAD, do), jnp.float32),
    )


_tc1_call = _make_tcmid(D, D)
_tc2_call = _make_tcmid(D, DO)


def _tc3_body(s_ref, p_ref, deg_ref, b_ref, wg_ref, a2_ref, g_ref, al_ref):
    dinv = _dinv_block(deg_ref, pl.program_id(0))
    h = (s_ref[0] + s_ref[1] + p_ref[...]) * dinv[:, None] + b_ref[...]
    g = jnp.dot(h, wg_ref[...], preferred_element_type=jnp.float32)
    g_ref[...] = g
    al_ref[...] = jnp.dot(g, a2_ref[...], preferred_element_type=jnp.float32)


_tc3_call = pl.pallas_call(
    _tc3_body,
    grid=(GRID,),
    in_specs=[
        pl.BlockSpec((NC, BLK, DO), lambda i: (0, i, 0)),
        pl.BlockSpec((BLK, DO), lambda i: (i, 0)),
        pl.BlockSpec((NC, NPAD), lambda i: (0, 0)),
        pl.BlockSpec((1, DO), lambda i: (0, 0)),
        pl.BlockSpec((DO, 4 * DO), lambda i: (0, 0)),
        pl.BlockSpec((4 * DO, 8), lambda i: (0, 0)),
    ],
    out_specs=[
        pl.BlockSpec((BLK, 4 * DO), lambda i: (i, 0)),
        pl.BlockSpec((BLK, 8), lambda i: (i, 0)),
    ],
    out_shape=[
        jax.ShapeDtypeStruct((NPAD, 4 * DO), jnp.float32),
        jax.ShapeDtypeStruct((NPAD, 8), jnp.float32),
    ],
)


def _tc4_body(a_ref, bg_ref, p1a_ref, p1b_ref, pb1_ref, h_ref, g1_ref, g2_ref):
    h = a_ref[0] + a_ref[1] + bg_ref[...]
    h_ref[...] = h
    g1_ref[...] = jnp.dot(h, p1a_ref[...],
                          preferred_element_type=jnp.float32) + pb1_ref[...]
    g2_ref[...] = jnp.dot(h, p1b_ref[...], preferred_element_type=jnp.float32)


_tc4_call = pl.pallas_call(
    _tc4_body,
    grid=(GRID,),
    in_specs=[
        pl.BlockSpec((NC, BLK, DO), lambda i: (0, i, 0)),
        pl.BlockSpec((1, DO), lambda i: (0, 0)),
        pl.BlockSpec((DO, D), lambda i: (0, 0)),
        pl.BlockSpec((DO, D), lambda i: (0, 0)),
        pl.BlockSpec((1, D), lambda i: (0, 0)),
    ],
    out_specs=[
        pl.BlockSpec((BLK, DO), lambda i: (i, 0)),
        pl.BlockSpec((BLK, D), lambda i: (i, 0)),
        pl.BlockSpec((BLK, D), lambda i: (i, 0)),
    ],
    out_shape=[
        jax.ShapeDtypeStruct((NPAD, DO), jnp.float32),
        jax.ShapeDtypeStruct((NPAD, D), jnp.float32),
        jax.ShapeDtypeStruct((NPAD, D), jnp.float32),
    ],
)


def kernel(x, edge_index, W1, b1, W2, b2, W3, b3, Wg, a_src, a_dst, bg,
           P1, pb1, P2, pb2):
    f32 = jnp.float32
    src = edge_index[0].astype(jnp.int32)
    dst = edge_index[1].astype(jnp.int32)

    pad1 = (N + (jnp.arange(EP1 - E) % (NPAD - N))).astype(jnp.int32)
    src1 = jnp.concatenate([src, pad1])
    dst1 = jnp.concatenate([dst, pad1])
    self_idx = jnp.arange(N, dtype=jnp.int32)
    pad2 = (N + (jnp.arange(EP2 - E - N) % (NPAD - N))).astype(jnp.int32)
    src2 = jnp.concatenate([src, self_idx, pad2])
    dst2 = jnp.concatenate([dst, self_idx, pad2])

    x_p = jnp.concatenate([x, jnp.zeros((NPAD - N, D), f32)])
    zeros_n = jnp.zeros((NPAD,), f32)
    zeros128 = jnp.zeros((NPAD, D), f32)
    zeros64 = jnp.zeros((NPAD, DO), f32)
    zeros4 = jnp.zeros((NPAD, 4), f32)

    rows = jnp.arange(4 * DO)
    hcol = rows // DO
    a2 = (jnp.zeros((4 * DO, 8), f32)
          .at[rows, hcol].set(a_src.reshape(-1))
          .at[rows, 4 + hcol].set(a_dst.reshape(-1)))

    deg2 = _deg_call(dst1, zeros_n)
    p1 = _tc0_call(x_p, deg2, W1)
    s1 = _gcn128(src1, dst1, p1, zeros128)
    p2 = _tc1_call(s1, p1, deg2, b1.reshape(1, D), W2)
    s2 = _gcn128(src1, dst1, p2, zeros128)
    p3 = _tc2_call(s2, p2, deg2, b2.reshape(1, D), W3)
    s3 = _gcn64(src1, dst1, p3, zeros64)
    g, al = _tc3_call(s3, p3, deg2, b3.reshape(1, DO), Wg, a2)

    gmax = jnp.max(al[:N, :4], axis=0) + jnp.max(al[:N, 4:], axis=0)
    gmax16 = jnp.zeros((16,), f32).at[:4].set(gmax)

    denom, exbuf = _gata_call(src2, dst2, al, gmax16, zeros4)
    gacc = _gatb_call(src2, dst2, exbuf, denom, g, zeros64)

    h, g1, g2 = _tc4_call(gacc, bg.reshape(1, DO), P1[:DO], P1[DO:],
                          pb1.reshape(1, D))

    pb2_16 = jnp.zeros((16,), f32).at[0].set(pb2[0])
    preds_p = _edge_call(src1, dst1, g1, g2, P2[:, 0], pb2_16)

    return (h[:N], preds_p[:E])


# speedup vs baseline: 9.3596x; 9.3596x over previous
import jax
import jax.numpy as jnp
from jax import lax
from jax.experimental import pallas as pl
from jax.experimental.pallas import tpu as pltpu
from jax.experimental.pallas import tpu_sc as plsc

N = 10000
NPAD = 10240
D = 128
DO = 64
E = 320000
NC, NS = 2, 16
NW = NC * NS
CH = 128
RPT = NPAD // NS


def _ceil_to(n, m):
    return ((n + m - 1) // m) * m


EP1 = _ceil_to(E, NW * CH)
EP2 = _ceil_to(E + N, NW * CH)
CH1 = EP1 // (NW * CH)
CH2 = EP2 // (NW * CH)
CH2A = EP2 // (NS * CH)

_MESH = plsc.VectorSubcoreMesh(
    core_axis_name="c", subcore_axis_name="s", num_cores=NC, num_subcores=NS
)


def _deg_body(dst_hbm, zeros_hbm, out_hbm, didx, ones_v, acc):
    c = lax.axis_index("c")
    s = lax.axis_index("s")
    w = c * NS + s
    r0 = s * RPT
    pltpu.sync_copy(zeros_hbm.at[pl.ds(r0, RPT)], acc.at[pl.ds(r0, RPT)])
    for g in range(8):
        ones_v[pl.ds(g * 16, 16)] = jnp.full((16,), 1.0, jnp.float32)
    plsc.subcore_barrier()

    def body(i, carry):
        base = (w * CH1 + i) * CH
        pltpu.sync_copy(dst_hbm.at[pl.ds(base, CH)], didx)
        pltpu.sync_copy(ones_v, acc.at[didx], add=True)
        return carry

    lax.fori_loop(0, CH1, body, 0)
    plsc.subcore_barrier()
    pltpu.sync_copy(acc.at[pl.ds(r0, RPT)], out_hbm.at[c, pl.ds(r0, RPT)])


_deg_call = pl.kernel(
    _deg_body,
    out_type=jax.ShapeDtypeStruct((NC, NPAD), jnp.float32),
    mesh=_MESH,
    scratch_types=[
        pltpu.VMEM((CH,), jnp.int32),
        pltpu.VMEM((CH,), jnp.float32),
        pltpu.VMEM_SHARED((NPAD,), jnp.float32),
    ],
)


def _gcn_body(src_hbm, dst_hbm, table_hbm, zeros_hbm, out_hbm, sidx, didx,
              rows, acc, sem):
    c = lax.axis_index("c")
    s = lax.axis_index("s")
    w = c * NS + s
    r0 = s * RPT
    pltpu.sync_copy(zeros_hbm.at[pl.ds(r0, RPT)], acc.at[pl.ds(r0, RPT)])
    plsc.subcore_barrier()

    def body(i, carry):
        base = (w * CH1 + i) * CH
        pltpu.sync_copy(src_hbm.at[pl.ds(base, CH)], sidx)
        pltpu.sync_copy(dst_hbm.at[pl.ds(base, CH)], didx)
        pltpu.async_copy(table_hbm.at[sidx], rows, sem).wait()
        pltpu.sync_copy(rows, acc.at[didx], add=True)
        return carry

    lax.fori_loop(0, CH1, body, 0)
    plsc.subcore_barrier()
    pltpu.sync_copy(acc.at[pl.ds(r0, RPT)], out_hbm.at[c, pl.ds(r0, RPT)])


def _make_gcn(dd):
    return pl.kernel(
        _gcn_body,
        out_type=jax.ShapeDtypeStruct((NC, NPAD, dd), jnp.float32),
        mesh=_MESH,
        scratch_types=[
            pltpu.VMEM((CH,), jnp.int32),
            pltpu.VMEM((CH,), jnp.int32),
            pltpu.VMEM((CH, dd), jnp.float32),
            pltpu.VMEM_SHARED((NPAD, dd), jnp.float32),
            pltpu.SemaphoreType.DMA,
        ],
    )


_gcn128 = _make_gcn(D)


RPT4 = NPAD * 4 // NS


def _gata_body(src_hbm, dst_hbm, al_hbm, gmax_hbm, zeros4_hbm, denom_hbm,
               ex_hbm, sidx, didx, tab, gmaxv, exv, idxb, dacc):
    c = lax.axis_index("c")
    s = lax.axis_index("s")

    @pl.when(c == 0)
    def _():
        r0 = s * RPT4
        pltpu.sync_copy(zeros4_hbm.at[pl.ds(r0, RPT4)], dacc.at[pl.ds(r0, RPT4)])
        pltpu.sync_copy(al_hbm, tab)
        pltpu.sync_copy(gmax_hbm, gmaxv)
        plsc.subcore_barrier()
        gv = gmaxv[pl.ds(0, 16)]

        def body(i, carry):
            base = (s * CH2A + i) * CH
            pltpu.sync_copy(src_hbm.at[pl.ds(base, CH)], sidx)
            pltpu.sync_copy(dst_hbm.at[pl.ds(base, CH)], didx)
            for g in range(8):
                sv = sidx[pl.ds(g * 16, 16)] * 8
                dv = didx[pl.ds(g * 16, 16)]
                dv8 = dv * 8
                dv4 = dv * 4
                for h in range(4):
                    a1 = plsc.load_gather(tab, [sv + h])
                    a2 = plsc.load_gather(tab, [dv8 + (4 + h)])
                    sc = a1 + a2
                    sc = jnp.maximum(sc, sc * 0.2) - gv[h]
                    exv[pl.ds(h * CH + g * 16, 16)] = jnp.exp(sc)
                    idxb[h, pl.ds(g * 16, 16)] = dv4 + h
            pltpu.sync_copy(exv, ex_hbm.at[pl.ds(base * 4, 4 * CH)])
            for h in range(4):
                pltpu.sync_copy(exv.at[pl.ds(h * CH, CH)],
                                dacc.at[idxb.at[h]], add=True)
            return carry

        lax.fori_loop(0, CH2A, body, 0)
        plsc.subcore_barrier()
        pltpu.sync_copy(dacc.at[pl.ds(r0, RPT4)], denom_hbm.at[pl.ds(r0, RPT4)])


_gata_call = pl.kernel(
    _gata_body,
    out_type=(
        jax.ShapeDtypeStruct((NPAD * 4,), jnp.float32),
        jax.ShapeDtypeStruct((EP2 * 4,), jnp.float32),
    ),
    mesh=_MESH,
    compiler_params=pltpu.CompilerParams(needs_layout_passes=False),
    scratch_types=[
        pltpu.VMEM((CH,), jnp.int32),
        pltpu.VMEM((CH,), jnp.int32),
        pltpu.VMEM((NPAD * 8,), jnp.float32),
        pltpu.VMEM((16,), jnp.float32),
        pltpu.VMEM((4 * CH,), jnp.float32),
        pltpu.VMEM((4, CH), jnp.int32),
        pltpu.VMEM_SHARED((NPAD * 4,), jnp.float32),
    ],
)


def _gata2_body(dst_hbm, ex_hbm, denom_hbm, w_hbm, didx, dtab, exv):
    c = lax.axis_index("c")
    s = lax.axis_index("s")
    w = c * NS + s
    pltpu.sync_copy(denom_hbm, dtab)

    def body(i, carry):
        base = (w * CH2 + i) * CH
        pltpu.sync_copy(dst_hbm.at[pl.ds(base, CH)], didx)
        pltpu.sync_copy(ex_hbm.at[pl.ds(base * 4, 4 * CH)], exv)

        def gbody(g, carry2):
            dv4 = didx[pl.ds(g * 16, 16)] * 4
            for h in range(4):
                dn = plsc.load_gather(dtab, [dv4 + h])
                e = exv[pl.ds(h * CH + g * 16, 16)]
                exv[pl.ds(h * CH + g * 16, 16)] = e / (dn + 1e-16) * 0.25
            return carry2

        lax.fori_loop(0, 8, gbody, 0)
        pltpu.sync_copy(exv, w_hbm.at[pl.ds(base * 4, 4 * CH)])
        return carry

    lax.fori_loop(0, CH2, body, 0)


_gata2_call = pl.kernel(
    _gata2_body,
    out_type=jax.ShapeDtypeStruct((EP2 * 4,), jnp.float32),
    mesh=_MESH,
    compiler_params=pltpu.CompilerParams(needs_layout_passes=False),
    scratch_types=[
        pltpu.VMEM((CH,), jnp.int32),
        pltpu.VMEM((NPAD * 4,), jnp.float32),
        pltpu.VMEM((4 * CH,), jnp.float32),
    ],
)


HC = CH // 2


def _gatb_body(src_hbm, dst_hbm, w_hbm, g_hbm, zeros128_hbm,
               out_hbm, sidx0, sidx1, didx0, didx1, exv, rows, orows,
               acc, sem):
    c = lax.axis_index("c")
    s = lax.axis_index("s")
    w = c * NS + s
    r0 = s * RPT
    pltpu.sync_copy(zeros128_hbm.at[pl.ds(r0, RPT)], acc.at[pl.ds(r0, RPT)])
    plsc.subcore_barrier()
    lanes = jnp.arange(16, dtype=jnp.int32)
    zv = jnp.zeros((16,), jnp.float32)

    def zbody(k, carry0):
        for j in range(4, 8):
            orows[k, pl.ds(j * 16, 16)] = zv
        return carry0

    lax.fori_loop(0, HC, zbody, 0)

    def body(i, carry):
        base = (w * CH2 + i) * CH
        pltpu.sync_copy(src_hbm.at[pl.ds(base, HC)], sidx0)
        pltpu.sync_copy(src_hbm.at[pl.ds(base + HC, HC)], sidx1)
        pltpu.sync_copy(dst_hbm.at[pl.ds(base, HC)], didx0)
        pltpu.sync_copy(dst_hbm.at[pl.ds(base + HC, HC)], didx1)
        pltpu.sync_copy(w_hbm.at[pl.ds(base * 4, 4 * CH)], exv)
        for j, (sx, dx) in enumerate(((sidx0, didx0), (sidx1, didx1))):
            pltpu.async_copy(g_hbm.at[sx], rows, sem).wait()

            def gbody(g, carry2, j=j):
                lid = lanes + g * 16
                ws = [exv[pl.ds(j * HC + g * 16 + h * CH, 16)]
                      for h in range(4)]
                for cc in range(DO):
                    o = ws[0] * plsc.load_gather(
                        rows, [lid, jnp.full((16,), cc, jnp.int32)])
                    for h in range(1, 4):
                        o = o + ws[h] * plsc.load_gather(
                            rows, [lid, jnp.full((16,), h * DO + cc,
                                                 jnp.int32)])
                    plsc.store_scatter(
                        orows, [lid, jnp.full((16,), cc, jnp.int32)], o)
                return carry2

            lax.fori_loop(0, 4, gbody, 0)
            pltpu.sync_copy(orows, acc.at[dx], add=True)
        return carry

    lax.fori_loop(0, CH2, body, 0)
    plsc.subcore_barrier()
    pltpu.sync_copy(acc.at[pl.ds(r0, RPT)], out_hbm.at[c, pl.ds(r0, RPT)])


_gatb_call = pl.kernel(
    _gatb_body,
    out_type=jax.ShapeDtypeStruct((NC, NPAD, D), jnp.float32),
    mesh=_MESH,
    compiler_params=pltpu.CompilerParams(needs_layout_passes=False),
    scratch_types=[
        pltpu.VMEM((HC,), jnp.int32),
        pltpu.VMEM((HC,), jnp.int32),
        pltpu.VMEM((HC,), jnp.int32),
        pltpu.VMEM((HC,), jnp.int32),
        pltpu.VMEM((4 * CH,), jnp.float32),
        pltpu.VMEM((HC, 256), jnp.float32),
        pltpu.VMEM((HC, D), jnp.float32),
        pltpu.VMEM_SHARED((NPAD, D), jnp.float32),
        pltpu.SemaphoreType.DMA,
    ],
)


def _edge_body(src_hbm, dst_hbm, g1_hbm, g2_hbm, p2_hbm, pb2_hbm, out_hbm,
               sidx, didx, r1, r2, p2v, pbv, predv, sem1, sem2):
    c = lax.axis_index("c")
    s = lax.axis_index("s")
    w = c * NS + s
    pltpu.sync_copy(p2_hbm, p2v)
    pltpu.sync_copy(pb2_hbm, pbv)
    lanes = jnp.arange(16, dtype=jnp.int32)

    def body(i, carry):
        base = (w * CH1 + i) * CH
        pltpu.sync_copy(src_hbm.at[pl.ds(base, CH)], sidx)
        pltpu.sync_copy(dst_hbm.at[pl.ds(base, CH)], didx)
        d1 = pltpu.async_copy(g1_hbm.at[sidx], r1, sem1)
        d2 = pltpu.async_copy(g2_hbm.at[didx], r2, sem2)
        d1.wait()
        d2.wait()
        p2regs = [p2v[pl.ds(j * 16, 16)] for j in range(8)]
        pb2s = pbv[pl.ds(0, 16)][0]

        def gbody(g, carry2):
            lid = lanes + g * 16
            acc = None
            for j in range(8):
                for u in range(16):
                    cn = jnp.full((16,), j * 16 + u, jnp.int32)
                    z = jnp.maximum(
                        plsc.load_gather(r1, [lid, cn])
                        + plsc.load_gather(r2, [lid, cn]), 0.0)
                    t = z * p2regs[j][u]
                    acc = t if acc is None else acc + t
            p = 1.0 / (1.0 + jnp.exp(-(acc + pb2s)))
            predv[pl.ds(g * 16, 16)] = p
            return carry2

        lax.fori_loop(0, 8, gbody, 0)
        pltpu.sync_copy(predv, out_hbm.at[pl.ds(base, CH)])
        return carry

    lax.fori_loop(0, CH1, body, 0)


_edge_call = pl.kernel(
    _edge_body,
    out_type=jax.ShapeDtypeStruct((EP1,), jnp.float32),
    mesh=_MESH,
    compiler_params=pltpu.CompilerParams(needs_layout_passes=False),
    scratch_types=[
        pltpu.VMEM((CH,), jnp.int32),
        pltpu.VMEM((CH,), jnp.int32),
        pltpu.VMEM((CH, D), jnp.float32),
        pltpu.VMEM((CH, D), jnp.float32),
        pltpu.VMEM((D,), jnp.float32),
        pltpu.VMEM((16,), jnp.float32),
        pltpu.VMEM((CH,), jnp.float32),
        pltpu.SemaphoreType.DMA,
        pltpu.SemaphoreType.DMA,
    ],
)


BLK = 256
GRID = NPAD // BLK


def _dinv_block(deg_ref, i):
    db = deg_ref[:, pl.ds(i * BLK, BLK)]
    d = db[0, :] + db[1, :] + 1.0
    return lax.rsqrt(jnp.maximum(d, 1.0))


def _tc0_body(x_ref, deg_ref, w_ref, o_ref):
    dinv = _dinv_block(deg_ref, pl.program_id(0))
    p = jnp.dot(x_ref[...], w_ref[...], preferred_element_type=jnp.float32)
    o_ref[...] = p * dinv[:, None]


_tc0_call = pl.pallas_call(
    _tc0_body,
    grid=(GRID,),
    in_specs=[
        pl.BlockSpec((BLK, D), lambda i: (i, 0)),
        pl.BlockSpec((NC, NPAD), lambda i: (0, 0)),
        pl.BlockSpec((D, D), lambda i: (0, 0)),
    ],
    out_specs=pl.BlockSpec((BLK, D), lambda i: (i, 0)),
    out_shape=jax.ShapeDtypeStruct((NPAD, D), jnp.float32),
)


def _tcmid_body(s_ref, p_ref, deg_ref, b_ref, w_ref, o_ref):
    dinv = _dinv_block(deg_ref, pl.program_id(0))
    h = s_ref[0] + s_ref[1] + p_ref[...]
    h = jnp.maximum(h * dinv[:, None] + b_ref[...], 0.0)
    o_ref[...] = jnp.dot(h, w_ref[...],
                         preferred_element_type=jnp.float32) * dinv[:, None]


def _make_tcmid(di, do):
    return pl.pallas_call(
        _tcmid_body,
        grid=(GRID,),
        in_specs=[
            pl.BlockSpec((NC, BLK, di), lambda i: (0, i, 0)),
            pl.BlockSpec((BLK, di), lambda i: (i, 0)),
            pl.BlockSpec((NC, NPAD), lambda i: (0, 0)),
            pl.BlockSpec((1, di), lambda i: (0, 0)),
            pl.BlockSpec((di, do), lambda i: (0, 0)),
        ],
        out_specs=pl.BlockSpec((BLK, do), lambda i: (i, 0)),
        out_shape=jax.ShapeDtypeStruct((NPAD, do), jnp.float32),
    )


_tc1_call = _make_tcmid(D, D)
_tc2_call = _make_tcmid(D, D)


def _tc3_body(s_ref, p_ref, deg_ref, b_ref, wg_ref, a2_ref, g_ref, al_ref):
    dinv = _dinv_block(deg_ref, pl.program_id(0))
    h = ((s_ref[0] + s_ref[1] + p_ref[...])[:, :DO] * dinv[:, None]
         + b_ref[...])
    g = jnp.dot(h, wg_ref[...], preferred_element_type=jnp.float32)
    g_ref[...] = g
    al_ref[...] = jnp.dot(g, a2_ref[...], preferred_element_type=jnp.float32)


_tc3_call = pl.pallas_call(
    _tc3_body,
    grid=(GRID,),
    in_specs=[
        pl.BlockSpec((NC, BLK, D), lambda i: (0, i, 0)),
        pl.BlockSpec((BLK, D), lambda i: (i, 0)),
        pl.BlockSpec((NC, NPAD), lambda i: (0, 0)),
        pl.BlockSpec((1, DO), lambda i: (0, 0)),
        pl.BlockSpec((DO, 4 * DO), lambda i: (0, 0)),
        pl.BlockSpec((4 * DO, 8), lambda i: (0, 0)),
    ],
    out_specs=[
        pl.BlockSpec((BLK, 4 * DO), lambda i: (i, 0)),
        pl.BlockSpec((BLK, 8), lambda i: (i, 0)),
    ],
    out_shape=[
        jax.ShapeDtypeStruct((NPAD, 4 * DO), jnp.float32),
        jax.ShapeDtypeStruct((NPAD, 8), jnp.float32),
    ],
)


def _tc4_body(a_ref, bg_ref, p1a_ref, p1b_ref, pb1_ref, h_ref, g1_ref, g2_ref):
    h = (a_ref[0] + a_ref[1])[:, :DO] + bg_ref[...]
    h_ref[...] = h
    g1_ref[...] = jnp.dot(h, p1a_ref[...],
                          preferred_element_type=jnp.float32) + pb1_ref[...]
    g2_ref[...] = jnp.dot(h, p1b_ref[...], preferred_element_type=jnp.float32)


_tc4_call = pl.pallas_call(
    _tc4_body,
    grid=(GRID,),
    in_specs=[
        pl.BlockSpec((NC, BLK, D), lambda i: (0, i, 0)),
        pl.BlockSpec((1, DO), lambda i: (0, 0)),
        pl.BlockSpec((DO, D), lambda i: (0, 0)),
        pl.BlockSpec((DO, D), lambda i: (0, 0)),
        pl.BlockSpec((1, D), lambda i: (0, 0)),
    ],
    out_specs=[
        pl.BlockSpec((BLK, DO), lambda i: (i, 0)),
        pl.BlockSpec((BLK, D), lambda i: (i, 0)),
        pl.BlockSpec((BLK, D), lambda i: (i, 0)),
    ],
    out_shape=[
        jax.ShapeDtypeStruct((NPAD, DO), jnp.float32),
        jax.ShapeDtypeStruct((NPAD, D), jnp.float32),
        jax.ShapeDtypeStruct((NPAD, D), jnp.float32),
    ],
)


def kernel(x, edge_index, W1, b1, W2, b2, W3, b3, Wg, a_src, a_dst, bg,
           P1, pb1, P2, pb2):
    f32 = jnp.float32
    src = edge_index[0].astype(jnp.int32)
    dst = edge_index[1].astype(jnp.int32)

    pad1 = (N + (jnp.arange(EP1 - E) % (NPAD - N))).astype(jnp.int32)
    src1 = jnp.concatenate([src, pad1])
    dst1 = jnp.concatenate([dst, pad1])
    self_idx = jnp.arange(N, dtype=jnp.int32)
    pad2 = (N + (jnp.arange(EP2 - E - N) % (NPAD - N))).astype(jnp.int32)
    src2 = jnp.concatenate([src, self_idx, pad2])
    dst2 = jnp.concatenate([dst, self_idx, pad2])

    x_p = jnp.concatenate([x, jnp.zeros((NPAD - N, D), f32)])
    zeros_n = jnp.zeros((NPAD,), f32)
    zeros128 = jnp.zeros((NPAD, D), f32)
    zeros4f = jnp.zeros((NPAD * 4,), f32)

    rows = jnp.arange(4 * DO)
    hcol = rows // DO
    a2 = (jnp.zeros((4 * DO, 8), f32)
          .at[rows, hcol].set(a_src.reshape(-1))
          .at[rows, 4 + hcol].set(a_dst.reshape(-1)))

    deg2 = _deg_call(dst1, zeros_n)
    p1 = _tc0_call(x_p, deg2, W1)
    s1 = _gcn128(src1, dst1, p1, zeros128)
    p2 = _tc1_call(s1, p1, deg2, b1.reshape(1, D), W2)
    s2 = _gcn128(src1, dst1, p2, zeros128)
    w3p = jnp.concatenate([W3, jnp.zeros((D, D - DO), f32)], axis=1)
    p3 = _tc2_call(s2, p2, deg2, b2.reshape(1, D), w3p)
    s3 = _gcn128(src1, dst1, p3, zeros128)
    g, al = _tc3_call(s3, p3, deg2, b3.reshape(1, DO), Wg, a2)

    gmax = jnp.max(al[:N, :4], axis=0) + jnp.max(al[:N, 4:], axis=0)
    gmax16 = jnp.zeros((16,), f32).at[:4].set(gmax)

    denom, exbuf = _gata_call(src2, dst2, al.reshape(-1), gmax16, zeros4f)
    wbuf = _gata2_call(dst2, exbuf, denom)
    gacc = _gatb_call(src2, dst2, wbuf, g, zeros128)

    h, g1, g2 = _tc4_call(gacc, bg.reshape(1, DO), P1[:DO], P1[DO:],
                          pb1.reshape(1, D))

    pb2_16 = jnp.zeros((16,), f32).at[0].set(pb2[0])
    preds_p = _edge_call(src1, dst1, g1, g2, P2[:, 0], pb2_16)

    return (h[:N], preds_p[:E])
